# baseline (speedup 1.0000x reference)
import functools

import jax
import jax.numpy as jnp
from jax import lax
from jax.experimental import pallas as pl
from jax.experimental.pallas import tpu as pltpu
from jax.experimental.pallas import tpu_sc as plsc

VOCAB = 1000000
DIM = 64
B = 16384
NEG = 20

NC = 2
NS = 16
NW = NC * NS
EX_PER_W = B // NW
E = 16
CHUNKS = EX_PER_W // E
ROWS_C = E * (1 + NEG)
GROUPS = ROWS_C // 16
NCHUNKS_TOT = B // E
IDX_PER_W = CHUNKS * ROWS_C


def _sc_scores(hidden2, idx_all, tab2):
    mesh = plsc.VectorSubcoreMesh(core_axis_name="c", subcore_axis_name="s")

    def row_gathers(tab_hbm, idx_buf, k, rows, hid, hid_hbm, sem):
        for (off, ln) in ((0, 128), (128, 128), (256, 80)):
            pltpu.async_copy(
                tab_hbm.at[idx_buf.at[pl.ds(k * ROWS_C + off, ln)]],
                rows.at[pl.ds(off, ln)], sem)
        pltpu.async_copy(hid_hbm.at[pl.ds(k * E, E)], hid, sem)

    def drain(tab_hbm, hid_hbm, rows, hid, sem):
        pltpu.make_async_copy(tab_hbm.at[pl.ds(0, ROWS_C)], rows, sem).wait()
        pltpu.make_async_copy(hid_hbm.at[pl.ds(0, E)], hid, sem).wait()

    @functools.partial(
        pl.kernel,
        mesh=mesh,
        compiler_params=pltpu.CompilerParams(needs_layout_passes=False),
        out_type=jax.ShapeDtypeStruct((B * (1 + NEG),), jnp.float32),
        scratch_types=[
            pltpu.VMEM((IDX_PER_W,), jnp.int32),
            pltpu.VMEM((ROWS_C, 2 * DIM), jnp.float32),
            pltpu.VMEM((ROWS_C, 2 * DIM), jnp.float32),
            pltpu.VMEM((E, 2 * DIM), jnp.float32),
            pltpu.VMEM((E, 2 * DIM), jnp.float32),
            pltpu.VMEM((ROWS_C,), jnp.float32),
            pltpu.VMEM((ROWS_C,), jnp.float32),
            pltpu.SemaphoreType.DMA,
            pltpu.SemaphoreType.DMA,
            pltpu.SemaphoreType.DMA,
            pltpu.SemaphoreType.DMA,
        ],
    )
    def body(hid_hbm, idx_hbm, tab_hbm, sc_out,
             idx_buf, rows0, rows1, hidb0, hidb1, sb0, sb1,
             sem0, sem1, ssem0, ssem1):
        wid = lax.axis_index("s") * NC + lax.axis_index("c")
        lanes = lax.iota(jnp.int32, 16)

        my_hid = hid_hbm.at[pl.ds(wid * EX_PER_W, EX_PER_W)]
        my_out = sc_out.at[pl.ds(wid * IDX_PER_W, IDX_PER_W)]

        pltpu.sync_copy(idx_hbm.at[pl.ds(wid * IDX_PER_W, IDX_PER_W)],
                        idx_buf)

        def compute(rows, hid, scores):
            def group_body(g, carry):
                cand = g * 16 + lanes
                el = jnp.where(cand < E, cand, (cand - E) // NEG)
                acc = jnp.zeros((16,), jnp.float32)
                for d in range(DIM):
                    dvec = lanes + d
                    acc = acc + (plsc.load_gather(rows, [cand, dvec]) *
                                 plsc.load_gather(hid, [el, dvec]))
                scores[pl.ds(g * 16, 16)] = acc
                return carry

            lax.fori_loop(0, GROUPS, group_body, jnp.int32(0))

        def halfstep(m, c, rows, hid, scores, sem, ssem, pf_c, pf_rows,
                     pf_hid, pf_sem):
            drain(tab_hbm, my_hid, rows, hid, sem)
            @pl.when(pf_c < CHUNKS)
            def _():
                row_gathers(tab_hbm, idx_buf, pf_c, pf_rows, pf_hid,
                            my_hid, pf_sem)
            @pl.when(m > 0)
            def _():
                _drain_scores(scores, ssem)
            compute(rows, hid, scores)
            pltpu.async_copy(scores, my_out.at[pl.ds(c * ROWS_C, ROWS_C)],
                             ssem)

        def _drain_scores(scores, ssem):
            pltpu.make_async_copy(my_out.at[pl.ds(0, ROWS_C)], scores,
                                  ssem).wait()

        row_gathers(tab_hbm, idx_buf, 0, rows0, hidb0, my_hid, sem0)

        def pair_body(m, carry):
            halfstep(m, 2 * m, rows0, hidb0, sb0, sem0, ssem0,
                     2 * m + 1, rows1, hidb1, sem1)
            halfstep(m, 2 * m + 1, rows1, hidb1, sb1, sem1, ssem1,
                     2 * m + 2, rows0, hidb0, sem0)
            return carry

        lax.fori_loop(0, CHUNKS // 2, pair_body, jnp.int32(0))

        _drain_scores(sb0, ssem0)
        _drain_scores(sb1, ssem1)

    return body(hidden2, idx_all, tab2)


def _tc_loss(scores2d):
    def tc_body(s_ref, out_ref):
        x = s_ref[...]
        r = lax.broadcasted_iota(jnp.int32, x.shape, 0)
        c = lax.broadcasted_iota(jnp.int32, x.shape, 1)
        q = (r * 128 + c) % ROWS_C
        is_pos = q < E
        m = jnp.where(is_pos, jnp.minimum(x, 0.0), -jnp.maximum(x, 0.0))
        t = m - jnp.log1p(jnp.exp(-jnp.abs(x)))
        w = jnp.where(is_pos, 1.0, 1.0 / NEG)
        out_ref[...] = jnp.broadcast_to(-jnp.sum(w * t), (1, 1))

    return pl.pallas_call(
        tc_body,
        out_shape=jax.ShapeDtypeStruct((1, 1), jnp.float32),
    )(scores2d)


def kernel(hidden_state, label_idxes, neg_idxes, out_table):
    lab = label_idxes.astype(jnp.int32).reshape(NCHUNKS_TOT, E)
    neg = neg_idxes.astype(jnp.int32).reshape(NCHUNKS_TOT, E * NEG)
    idx_all = jnp.concatenate([lab, neg], axis=1).reshape(-1)
    tab2 = jnp.concatenate([out_table, out_table], axis=1)
    hid2 = jnp.concatenate([hidden_state, hidden_state], axis=1)
    scores = _sc_scores(hid2, idx_all, tab2)
    return _tc_loss(scores.reshape(B * (1 + NEG) // 128, 128))[0, 0]


# speedup vs baseline: 1.1097x; 1.1097x over previous
import functools

import jax
import jax.numpy as jnp
from jax import lax
from jax.experimental import pallas as pl
from jax.experimental.pallas import tpu as pltpu
from jax.experimental.pallas import tpu_sc as plsc

VOCAB = 1000000
DIM = 64
B = 16384
NEG = 20

NC = 2
NS = 16
NW = NC * NS
EX_PER_W = B // NW
E = 16
CHUNKS = EX_PER_W // E
ROWS_C = E * (1 + NEG)
GROUPS = ROWS_C // 16
NCHUNKS_TOT = B // E
IDX_PER_W = CHUNKS * ROWS_C


def _sc_scores(hidden2, idx_all, idxf_all, tab2):
    mesh = plsc.VectorSubcoreMesh(core_axis_name="c", subcore_axis_name="s")

    def row_gathers(tab_hbm, idxf_buf, k, rows, hid, hid_hbm, sem):
        for (off, ln) in ((0, 128), (128, 128), (256, 80)):
            pltpu.async_copy(
                tab_hbm.at[idxf_buf.at[pl.ds(k * ROWS_C + off, ln)]],
                rows.at[pl.ds(off, ln)], sem)
        pltpu.async_copy(hid_hbm.at[pl.ds(k * E, E)], hid, sem)

    def drain(tab_hbm, hid_hbm, rows, hid, sem):
        pltpu.make_async_copy(tab_hbm.at[pl.ds(0, ROWS_C)], rows, sem).wait()
        pltpu.make_async_copy(hid_hbm.at[pl.ds(0, E)], hid, sem).wait()

    @functools.partial(
        pl.kernel,
        mesh=mesh,
        compiler_params=pltpu.CompilerParams(needs_layout_passes=False),
        out_type=jax.ShapeDtypeStruct((B * (1 + NEG),), jnp.float32),
        scratch_types=[
            pltpu.VMEM((IDX_PER_W,), jnp.int32),
            pltpu.VMEM((IDX_PER_W,), jnp.int32),
            pltpu.VMEM((ROWS_C, 2 * DIM), jnp.float32),
            pltpu.VMEM((ROWS_C, 2 * DIM), jnp.float32),
            pltpu.VMEM((E, 2 * DIM), jnp.float32),
            pltpu.VMEM((E, 2 * DIM), jnp.float32),
            pltpu.VMEM((ROWS_C,), jnp.float32),
            pltpu.VMEM((ROWS_C,), jnp.float32),
            pltpu.SemaphoreType.DMA,
            pltpu.SemaphoreType.DMA,
            pltpu.SemaphoreType.DMA,
            pltpu.SemaphoreType.DMA,
        ],
    )
    def body(hid_hbm, idx_hbm, idxf_hbm, tab_hbm, sc_out,
             idx_buf, idxf_buf, rows0, rows1, hidb0, hidb1, sb0, sb1,
             sem0, sem1, ssem0, ssem1):
        wid = lax.axis_index("s") * NC + lax.axis_index("c")
        lanes = lax.iota(jnp.int32, 16)

        my_hid = hid_hbm.at[pl.ds(wid * EX_PER_W, EX_PER_W)]
        my_out = sc_out.at[pl.ds(wid * IDX_PER_W, IDX_PER_W)]

        pltpu.sync_copy(idx_hbm.at[pl.ds(wid * IDX_PER_W, IDX_PER_W)],
                        idx_buf)
        pltpu.sync_copy(idxf_hbm.at[pl.ds(wid * IDX_PER_W, IDX_PER_W)],
                        idxf_buf)

        def compute(k, rows, hid, scores):
            def group_body(g, carry):
                cand = g * 16 + lanes
                el = jnp.where(cand < E, cand, (cand - E) // NEG)
                half = plsc.load_gather(idx_buf, [k * ROWS_C + cand]) & 1
                hoff = half * DIM
                acc = jnp.zeros((16,), jnp.float32)
                for d in range(DIM):
                    dvec = lanes + d
                    dvec64 = dvec & (DIM - 1)
                    acc = acc + (
                        plsc.load_gather(rows, [cand, hoff + dvec64]) *
                        plsc.load_gather(hid, [el, dvec]))
                scores[pl.ds(g * 16, 16)] = acc
                return carry

            lax.fori_loop(0, GROUPS, group_body, jnp.int32(0))

        def halfstep(m, c, rows, hid, scores, sem, ssem, pf_c, pf_rows,
                     pf_hid, pf_sem):
            drain(tab_hbm, my_hid, rows, hid, sem)
            @pl.when(pf_c < CHUNKS)
            def _():
                row_gathers(tab_hbm, idxf_buf, pf_c, pf_rows, pf_hid,
                            my_hid, pf_sem)
            @pl.when(m > 0)
            def _():
                _drain_scores(scores, ssem)
            compute(c, rows, hid, scores)
            pltpu.async_copy(scores, my_out.at[pl.ds(c * ROWS_C, ROWS_C)],
                             ssem)

        def _drain_scores(scores, ssem):
            pltpu.make_async_copy(my_out.at[pl.ds(0, ROWS_C)], scores,
                                  ssem).wait()

        row_gathers(tab_hbm, idxf_buf, 0, rows0, hidb0, my_hid, sem0)

        def pair_body(m, carry):
            halfstep(m, 2 * m, rows0, hidb0, sb0, sem0, ssem0,
                     2 * m + 1, rows1, hidb1, sem1)
            halfstep(m, 2 * m + 1, rows1, hidb1, sb1, sem1, ssem1,
                     2 * m + 2, rows0, hidb0, sem0)
            return carry

        lax.fori_loop(0, CHUNKS // 2, pair_body, jnp.int32(0))

        _drain_scores(sb0, ssem0)
        _drain_scores(sb1, ssem1)

    return body(hidden2, idx_all, idxf_all, tab2)


def _tc_loss(scores2d):
    def tc_body(s_ref, out_ref):
        x = s_ref[...]
        r = lax.broadcasted_iota(jnp.int32, x.shape, 0)
        c = lax.broadcasted_iota(jnp.int32, x.shape, 1)
        q = (r * 128 + c) % ROWS_C
        is_pos = q < E
        m = jnp.where(is_pos, jnp.minimum(x, 0.0), -jnp.maximum(x, 0.0))
        t = m - jnp.log1p(jnp.exp(-jnp.abs(x)))
        w = jnp.where(is_pos, 1.0, 1.0 / NEG)
        out_ref[...] = jnp.broadcast_to(-jnp.sum(w * t), (1, 1))

    return pl.pallas_call(
        tc_body,
        out_shape=jax.ShapeDtypeStruct((1, 1), jnp.float32),
    )(scores2d)


def kernel(hidden_state, label_idxes, neg_idxes, out_table):
    lab = label_idxes.astype(jnp.int32).reshape(NCHUNKS_TOT, E)
    neg = neg_idxes.astype(jnp.int32).reshape(NCHUNKS_TOT, E * NEG)
    idx_all = jnp.concatenate([lab, neg], axis=1).reshape(-1)
    idxf_all = idx_all >> 1
    tab2 = out_table.reshape(VOCAB // 2, 2 * DIM)
    hid2 = jnp.concatenate([hidden_state, hidden_state], axis=1)
    scores = _sc_scores(hid2, idx_all, idxf_all, tab2)
    return _tc_loss(scores.reshape(B * (1 + NEG) // 128, 128))[0, 0]
